# Initial kernel scaffold; baseline (speedup 1.0000x reference)
import jax
import jax.numpy as jnp
from jax.experimental import pallas as pl


def kernel(x, edge_index, mut_mask, wt_idx, mut_idx, W1, b1, W2, b2, aa_emb, Wh1, bh1, Wh2, bh2):
    raise NotImplementedError("write your pallas kernel here")


# speedup vs baseline: 6.2303x; 6.2303x over previous
import functools

import jax
import jax.numpy as jnp
from jax import lax
from jax.experimental import pallas as pl
from jax.experimental.pallas import tpu as pltpu
from jax.experimental.pallas import tpu_sc as plsc

N_NODES = 10000
N_EDGES = 160000
DIM = 256
HALF = 128

NPAD = 10112
JUNK_ROW = 10016
EPAD = 163840
SEG_CHUNKS = 80
DEG_CHUNKS = 40
CHUNK = 128
ROWS_PER_TILE = NPAD // 16


def _deg_body(cols_hbm, ones_hbm, zeros_hbm, deg_hbm, colv, ones_v, acc, sem):
    c = lax.axis_index("c")
    s = lax.axis_index("s")
    tile = c * 16 + s
    pltpu.sync_copy(cols_hbm.at[tile], colv)
    pltpu.sync_copy(ones_hbm, ones_v)
    pltpu.sync_copy(
        zeros_hbm.at[pl.ds(s * ROWS_PER_TILE, ROWS_PER_TILE)],
        acc.at[pl.ds(s * ROWS_PER_TILE, ROWS_PER_TILE)],
    )
    plsc.subcore_barrier()

    @pl.loop(0, DEG_CHUNKS)
    def _(j):
        pltpu.sync_copy(ones_v, acc.at[colv.at[j]], add=True)

    plsc.subcore_barrier()
    pltpu.sync_copy(
        acc.at[pl.ds(s * ROWS_PER_TILE, ROWS_PER_TILE)],
        deg_hbm.at[c, pl.ds(s * ROWS_PER_TILE, ROWS_PER_TILE)],
    )


_deg_kernel = pl.kernel(
    _deg_body,
    out_type=jax.ShapeDtypeStruct((2, NPAD, HALF), jnp.float32),
    mesh=plsc.VectorSubcoreMesh(core_axis_name="c", subcore_axis_name="s"),
    scratch_types=[
        pltpu.VMEM((DEG_CHUNKS, CHUNK), jnp.int32),
        pltpu.VMEM((CHUNK, HALF), jnp.float32),
        pltpu.VMEM_SHARED((NPAD, HALF), jnp.float32),
        pltpu.SemaphoreType.DMA,
    ],
)


def _seg_body(s_hbm, rows_hbm, cols_hbm, zeros_hbm, t_hbm,
              rowv, colv, buf, acc, sem):
    c = lax.axis_index("c")
    s = lax.axis_index("s")
    tile = c * 16 + s
    pltpu.sync_copy(rows_hbm.at[tile], rowv)
    pltpu.sync_copy(cols_hbm.at[s], colv)
    pltpu.sync_copy(
        zeros_hbm.at[pl.ds(s * ROWS_PER_TILE, ROWS_PER_TILE)],
        acc.at[pl.ds(s * ROWS_PER_TILE, ROWS_PER_TILE)],
    )
    plsc.subcore_barrier()

    @pl.loop(0, SEG_CHUNKS)
    def _(j):
        pltpu.async_copy(s_hbm.at[rowv.at[j]], buf, sem).wait()
        pltpu.sync_copy(buf, acc.at[colv.at[j]], add=True)

    plsc.subcore_barrier()
    pltpu.sync_copy(
        acc.at[pl.ds(s * ROWS_PER_TILE, ROWS_PER_TILE)],
        t_hbm.at[c, pl.ds(s * ROWS_PER_TILE, ROWS_PER_TILE)],
    )


_seg_kernel = pl.kernel(
    _seg_body,
    out_type=jax.ShapeDtypeStruct((2, NPAD, HALF), jnp.float32),
    mesh=plsc.VectorSubcoreMesh(core_axis_name="c", subcore_axis_name="s"),
    scratch_types=[
        pltpu.VMEM((SEG_CHUNKS, CHUNK), jnp.int32),
        pltpu.VMEM((SEG_CHUNKS, CHUNK), jnp.int32),
        pltpu.VMEM((CHUNK, HALF), jnp.float32),
        pltpu.VMEM_SHARED((NPAD, HALF), jnp.float32),
        pltpu.SemaphoreType.DMA,
    ],
)


def _dis_from(deg_ref):
    deg = deg_ref[0, :, 0:1] + deg_ref[1, :, 0:1] + 1.0
    return lax.rsqrt(deg)


def _mm1_body(deg_ref, x_ref, w_ref, o_ref):
    dis = _dis_from(deg_ref)
    o_ref[...] = dis * jnp.dot(x_ref[...], w_ref[...],
                               preferred_element_type=jnp.float32)


def _comb_body(deg_ref, t_ref, s_ref, b_ref, w_ref, o_ref):
    dis = _dis_from(deg_ref)
    ha = jnp.maximum(dis * (t_ref[0] + s_ref[0]) + b_ref[:, :HALF], 0.0)
    hb = jnp.maximum(dis * (t_ref[1] + s_ref[1]) + b_ref[:, HALF:], 0.0)
    h = jnp.concatenate([ha, hb], axis=1)
    o_ref[...] = dis * jnp.dot(h, w_ref[...], preferred_element_type=jnp.float32)


def _fin_body(wt_ref, mut_ref, deg_ref, t_ref, s_ref, b_ref, mask_ref,
              aa_ref, wh1_ref, bh1_ref, wh2_ref, bh2_ref, o_ref, zacc):
    i = pl.program_id(0)
    dis = _dis_from(deg_ref)
    ha = jnp.maximum(dis * (t_ref[0] + s_ref[0]) + b_ref[:, :HALF], 0.0)
    hb = jnp.maximum(dis * (t_ref[1] + s_ref[1]) + b_ref[:, HALF:], 0.0)
    h = jnp.concatenate([ha, hb], axis=1)
    part = jnp.dot(mask_ref[pl.ds(i, 1), :], h,
                   preferred_element_type=jnp.float32)

    @pl.when(i == 0)
    def _():
        zacc[...] = part

    @pl.when(i != 0)
    def _():
        zacc[...] = zacc[...] + part

    @pl.when(i == pl.num_programs(0) - 1)
    def _():
        z = zacc[...]
        wt = wt_ref[0]
        mut = mut_ref[0]
        delta = aa_ref[pl.ds(mut, 1), :] - aa_ref[pl.ds(wt, 1), :]
        hid = (jnp.dot(z, wh1_ref[:DIM, :], preferred_element_type=jnp.float32)
               + jnp.dot(delta, wh1_ref[DIM:, :],
                         preferred_element_type=jnp.float32)
               + bh1_ref[...])
        hid = jnp.maximum(hid, 0.0)
        o_ref[...] = jnp.dot(hid, wh2_ref[...],
                             preferred_element_type=jnp.float32) + bh2_ref[...]


_BLK = 1000


def _mm1_call(deg2, x, w1):
    return pl.pallas_call(
        _mm1_body,
        grid=(2, 10),
        in_specs=[
            pl.BlockSpec((2, _BLK, HALF), lambda j, i: (0, i, 0)),
            pl.BlockSpec((_BLK, DIM), lambda j, i: (i, 0)),
            pl.BlockSpec((DIM, HALF), lambda j, i: (0, j)),
        ],
        out_specs=pl.BlockSpec((_BLK, HALF), lambda j, i: (j * 10 + i, 0)),
        out_shape=jax.ShapeDtypeStruct((2 * N_NODES, HALF), jnp.float32),
    )(deg2, x, w1)


def _comb_call(deg2, t, s_r, b, w2):
    return pl.pallas_call(
        _comb_body,
        grid=(2, 10),
        in_specs=[
            pl.BlockSpec((2, _BLK, HALF), lambda j, i: (0, i, 0)),
            pl.BlockSpec((2, _BLK, HALF), lambda j, i: (0, i, 0)),
            pl.BlockSpec((2, _BLK, HALF), lambda j, i: (0, i, 0)),
            pl.BlockSpec((1, DIM), lambda j, i: (0, 0)),
            pl.BlockSpec((DIM, HALF), lambda j, i: (0, j)),
        ],
        out_specs=pl.BlockSpec((_BLK, HALF), lambda j, i: (j * 10 + i, 0)),
        out_shape=jax.ShapeDtypeStruct((2 * N_NODES, HALF), jnp.float32),
    )(deg2, t, s_r, b, w2)


def _fin_call(wt_idx, mut_idx, deg2, t, s_r, b, mask, aa, wh1, bh1, wh2, bh2):
    return pl.pallas_call(
        _fin_body,
        grid=(10,),
        in_specs=[
            pl.BlockSpec(memory_space=pltpu.MemorySpace.SMEM),
            pl.BlockSpec(memory_space=pltpu.MemorySpace.SMEM),
            pl.BlockSpec((2, _BLK, HALF), lambda i: (0, i, 0)),
            pl.BlockSpec((2, _BLK, HALF), lambda i: (0, i, 0)),
            pl.BlockSpec((2, _BLK, HALF), lambda i: (0, i, 0)),
            pl.BlockSpec((1, DIM), lambda i: (0, 0)),
            pl.BlockSpec((10, _BLK), lambda i: (0, 0)),
            pl.BlockSpec((20, 64), lambda i: (0, 0)),
            pl.BlockSpec((DIM + 64, DIM), lambda i: (0, 0)),
            pl.BlockSpec((1, DIM), lambda i: (0, 0)),
            pl.BlockSpec((DIM, 1), lambda i: (0, 0)),
            pl.BlockSpec((1, 1), lambda i: (0, 0)),
        ],
        out_specs=pl.BlockSpec((1, 1), lambda i: (0, 0)),
        out_shape=jax.ShapeDtypeStruct((1, 1), jnp.float32),
        scratch_shapes=[pltpu.VMEM((1, DIM), jnp.float32)],
    )(wt_idx, mut_idx, deg2, t, s_r, b, mask, aa, wh1, bh1, wh2, bh2)


def kernel(x, edge_index, mut_mask, wt_idx, mut_idx,
           W1, b1, W2, b2, aa_emb, Wh1, bh1, Wh2, bh2):
    row = edge_index[0].astype(jnp.int32)
    col = edge_index[1].astype(jnp.int32)
    pad = EPAD - N_EDGES
    rowp = jnp.concatenate([row, jnp.zeros((pad,), jnp.int32)])
    colp = jnp.concatenate([col, jnp.full((pad,), JUNK_ROW, jnp.int32)])

    r16 = rowp.reshape(16, SEG_CHUNKS, CHUNK)
    rows_seg = jnp.concatenate([r16, r16 + N_NODES], axis=0)
    cols_seg = colp.reshape(16, SEG_CHUNKS, CHUNK)
    cols_deg = colp.reshape(32, DEG_CHUNKS, CHUNK)

    ones128 = jnp.ones((CHUNK, HALF), jnp.float32)
    zeros_big = jnp.zeros((NPAD, HALF), jnp.float32)

    deg2 = _deg_kernel(cols_deg, ones128, zeros_big)

    s1 = _mm1_call(deg2, x, W1)
    t1 = _seg_kernel(s1, rows_seg, cols_seg, zeros_big)
    s1_r = s1.reshape(2, N_NODES, HALF)

    s2 = _comb_call(deg2, t1, s1_r, b1.reshape(1, DIM), W2)
    t2 = _seg_kernel(s2, rows_seg, cols_seg, zeros_big)
    s2_r = s2.reshape(2, N_NODES, HALF)

    out = _fin_call(wt_idx.astype(jnp.int32), mut_idx.astype(jnp.int32),
                    deg2, t2, s2_r, b2.reshape(1, DIM),
                    mut_mask.reshape(10, _BLK), aa_emb, Wh1,
                    bh1.reshape(1, DIM), Wh2, bh2.reshape(1, 1))
    return out[0, 0]


# speedup vs baseline: 7.0251x; 1.1276x over previous
import functools

import jax
import jax.numpy as jnp
from jax import lax
from jax.experimental import pallas as pl
from jax.experimental.pallas import tpu as pltpu
from jax.experimental.pallas import tpu_sc as plsc

N_NODES = 10000
N_EDGES = 160000
DIM = 256
HALF = 128

NPAD = 10112
JUNK_ROW = 10016
EPAD = 163840
SEG_CHUNKS = 80
DEG_CHUNKS = 40
CHUNK = 128
ROWS_PER_TILE = NPAD // 16


def _deg_body(cols_hbm, ones_hbm, zeros_hbm, deg_hbm, colv, ones_v, acc, sem):
    c = lax.axis_index("c")
    s = lax.axis_index("s")
    tile = c * 16 + s
    pltpu.sync_copy(cols_hbm.at[tile], colv)
    pltpu.sync_copy(ones_hbm, ones_v)
    pltpu.sync_copy(
        zeros_hbm.at[pl.ds(s * ROWS_PER_TILE, ROWS_PER_TILE)],
        acc.at[pl.ds(s * ROWS_PER_TILE, ROWS_PER_TILE)],
    )
    plsc.subcore_barrier()

    @pl.loop(0, DEG_CHUNKS)
    def _(j):
        pltpu.sync_copy(ones_v, acc.at[colv.at[j]], add=True)

    plsc.subcore_barrier()
    pltpu.sync_copy(
        acc.at[pl.ds(s * ROWS_PER_TILE, ROWS_PER_TILE)],
        deg_hbm.at[c, pl.ds(s * ROWS_PER_TILE, ROWS_PER_TILE)],
    )


_deg_kernel = pl.kernel(
    _deg_body,
    out_type=jax.ShapeDtypeStruct((2, NPAD, HALF), jnp.float32),
    mesh=plsc.VectorSubcoreMesh(core_axis_name="c", subcore_axis_name="s"),
    scratch_types=[
        pltpu.VMEM((DEG_CHUNKS, CHUNK), jnp.int32),
        pltpu.VMEM((CHUNK, HALF), jnp.float32),
        pltpu.VMEM_SHARED((NPAD, HALF), jnp.float32),
        pltpu.SemaphoreType.DMA,
    ],
)


NBUF = 2
N_PASS = 2
PASS_CHUNKS = SEG_CHUNKS // N_PASS


def _seg_body(s_hbm, rows_hbm, cols_hbm, zeros_hbm, t_hbm,
              rowv, colv, buf0, buf1, acc, sem0, sem1):
    c = lax.axis_index("c")
    s = lax.axis_index("s")
    tile = c * 16 + s
    bufs = (buf0, buf1)
    sems = (sem0, sem1)
    pltpu.sync_copy(
        zeros_hbm.at[pl.ds(s * ROWS_PER_TILE, ROWS_PER_TILE)],
        acc.at[pl.ds(s * ROWS_PER_TILE, ROWS_PER_TILE)],
    )
    plsc.subcore_barrier()

    @pl.loop(0, N_PASS)
    def _(p):
        pltpu.sync_copy(
            rows_hbm.at[tile, pl.ds(p * PASS_CHUNKS, PASS_CHUNKS)], rowv)
        pltpu.sync_copy(
            cols_hbm.at[s, pl.ds(p * PASS_CHUNKS, PASS_CHUNKS)], colv)

        for b in range(NBUF):
            pltpu.async_copy(s_hbm.at[rowv.at[b]], bufs[b], sems[b])

        @pl.loop(0, PASS_CHUNKS - NBUF, step=NBUF)
        def _(j):
            for b in range(NBUF):
                pltpu.make_async_copy(
                    s_hbm.at[rowv.at[b]], bufs[b], sems[b]).wait()
                pltpu.sync_copy(bufs[b], acc.at[colv.at[j + b]], add=True)
                pltpu.async_copy(
                    s_hbm.at[rowv.at[j + NBUF + b]], bufs[b], sems[b])

        for b in range(NBUF):
            pltpu.make_async_copy(
                s_hbm.at[rowv.at[b]], bufs[b], sems[b]).wait()
            pltpu.sync_copy(bufs[b], acc.at[colv.at[PASS_CHUNKS - NBUF + b]],
                            add=True)

    plsc.subcore_barrier()
    pltpu.sync_copy(
        acc.at[pl.ds(s * ROWS_PER_TILE, ROWS_PER_TILE)],
        t_hbm.at[c, pl.ds(s * ROWS_PER_TILE, ROWS_PER_TILE)],
    )


_seg_kernel = pl.kernel(
    _seg_body,
    out_type=jax.ShapeDtypeStruct((2, NPAD, HALF), jnp.float32),
    mesh=plsc.VectorSubcoreMesh(core_axis_name="c", subcore_axis_name="s"),
    scratch_types=[
        pltpu.VMEM((PASS_CHUNKS, CHUNK), jnp.int32),
        pltpu.VMEM((PASS_CHUNKS, CHUNK), jnp.int32),
        pltpu.VMEM((CHUNK, HALF), jnp.float32),
        pltpu.VMEM((CHUNK, HALF), jnp.float32),
        pltpu.VMEM_SHARED((NPAD, HALF), jnp.float32),
        pltpu.SemaphoreType.DMA,
        pltpu.SemaphoreType.DMA,
    ],
)


def _dis_from(deg_ref):
    deg = deg_ref[0, :, 0:1] + deg_ref[1, :, 0:1] + 1.0
    return lax.rsqrt(deg)


def _mm1_body(deg_ref, x_ref, w_ref, o_ref):
    dis = _dis_from(deg_ref)
    o_ref[...] = dis * jnp.dot(x_ref[...], w_ref[...],
                               preferred_element_type=jnp.float32)


def _comb_body(deg_ref, t_ref, s_ref, b_ref, w_ref, o_ref):
    dis = _dis_from(deg_ref)
    ha = jnp.maximum(dis * (t_ref[0] + s_ref[0]) + b_ref[:, :HALF], 0.0)
    hb = jnp.maximum(dis * (t_ref[1] + s_ref[1]) + b_ref[:, HALF:], 0.0)
    h = jnp.concatenate([ha, hb], axis=1)
    o_ref[...] = dis * jnp.dot(h, w_ref[...], preferred_element_type=jnp.float32)


def _fin_body(wt_ref, mut_ref, deg_ref, t_ref, s_ref, b_ref, mask_ref,
              aa_ref, wh1_ref, bh1_ref, wh2_ref, bh2_ref, o_ref, zacc):
    i = pl.program_id(0)
    dis = _dis_from(deg_ref)
    ha = jnp.maximum(dis * (t_ref[0] + s_ref[0]) + b_ref[:, :HALF], 0.0)
    hb = jnp.maximum(dis * (t_ref[1] + s_ref[1]) + b_ref[:, HALF:], 0.0)
    h = jnp.concatenate([ha, hb], axis=1)
    part = jnp.dot(mask_ref[pl.ds(i, 1), :], h,
                   preferred_element_type=jnp.float32)

    @pl.when(i == 0)
    def _():
        zacc[...] = part

    @pl.when(i != 0)
    def _():
        zacc[...] = zacc[...] + part

    @pl.when(i == pl.num_programs(0) - 1)
    def _():
        z = zacc[...]
        wt = wt_ref[0]
        mut = mut_ref[0]
        delta = aa_ref[pl.ds(mut, 1), :] - aa_ref[pl.ds(wt, 1), :]
        hid = (jnp.dot(z, wh1_ref[:DIM, :], preferred_element_type=jnp.float32)
               + jnp.dot(delta, wh1_ref[DIM:, :],
                         preferred_element_type=jnp.float32)
               + bh1_ref[...])
        hid = jnp.maximum(hid, 0.0)
        o_ref[...] = jnp.dot(hid, wh2_ref[...],
                             preferred_element_type=jnp.float32) + bh2_ref[...]


_BLK = 1000


def _mm1_call(deg2, x, w1):
    return pl.pallas_call(
        _mm1_body,
        grid=(2, 10),
        in_specs=[
            pl.BlockSpec((2, _BLK, HALF), lambda j, i: (0, i, 0)),
            pl.BlockSpec((_BLK, DIM), lambda j, i: (i, 0)),
            pl.BlockSpec((DIM, HALF), lambda j, i: (0, j)),
        ],
        out_specs=pl.BlockSpec((_BLK, HALF), lambda j, i: (j * 10 + i, 0)),
        out_shape=jax.ShapeDtypeStruct((2 * N_NODES, HALF), jnp.float32),
    )(deg2, x, w1)


def _comb_call(deg2, t, s_r, b, w2):
    return pl.pallas_call(
        _comb_body,
        grid=(2, 10),
        in_specs=[
            pl.BlockSpec((2, _BLK, HALF), lambda j, i: (0, i, 0)),
            pl.BlockSpec((2, _BLK, HALF), lambda j, i: (0, i, 0)),
            pl.BlockSpec((2, _BLK, HALF), lambda j, i: (0, i, 0)),
            pl.BlockSpec((1, DIM), lambda j, i: (0, 0)),
            pl.BlockSpec((DIM, HALF), lambda j, i: (0, j)),
        ],
        out_specs=pl.BlockSpec((_BLK, HALF), lambda j, i: (j * 10 + i, 0)),
        out_shape=jax.ShapeDtypeStruct((2 * N_NODES, HALF), jnp.float32),
    )(deg2, t, s_r, b, w2)


def _fin_call(wt_idx, mut_idx, deg2, t, s_r, b, mask, aa, wh1, bh1, wh2, bh2):
    return pl.pallas_call(
        _fin_body,
        grid=(10,),
        in_specs=[
            pl.BlockSpec(memory_space=pltpu.MemorySpace.SMEM),
            pl.BlockSpec(memory_space=pltpu.MemorySpace.SMEM),
            pl.BlockSpec((2, _BLK, HALF), lambda i: (0, i, 0)),
            pl.BlockSpec((2, _BLK, HALF), lambda i: (0, i, 0)),
            pl.BlockSpec((2, _BLK, HALF), lambda i: (0, i, 0)),
            pl.BlockSpec((1, DIM), lambda i: (0, 0)),
            pl.BlockSpec((10, _BLK), lambda i: (0, 0)),
            pl.BlockSpec((20, 64), lambda i: (0, 0)),
            pl.BlockSpec((DIM + 64, DIM), lambda i: (0, 0)),
            pl.BlockSpec((1, DIM), lambda i: (0, 0)),
            pl.BlockSpec((DIM, 1), lambda i: (0, 0)),
            pl.BlockSpec((1, 1), lambda i: (0, 0)),
        ],
        out_specs=pl.BlockSpec((1, 1), lambda i: (0, 0)),
        out_shape=jax.ShapeDtypeStruct((1, 1), jnp.float32),
        scratch_shapes=[pltpu.VMEM((1, DIM), jnp.float32)],
    )(wt_idx, mut_idx, deg2, t, s_r, b, mask, aa, wh1, bh1, wh2, bh2)


def kernel(x, edge_index, mut_mask, wt_idx, mut_idx,
           W1, b1, W2, b2, aa_emb, Wh1, bh1, Wh2, bh2):
    row = edge_index[0].astype(jnp.int32)
    col = edge_index[1].astype(jnp.int32)
    pad = EPAD - N_EDGES
    rowp = jnp.concatenate([row, jnp.zeros((pad,), jnp.int32)])
    colp = jnp.concatenate([col, jnp.full((pad,), JUNK_ROW, jnp.int32)])

    r16 = rowp.reshape(16, SEG_CHUNKS, CHUNK)
    rows_seg = jnp.concatenate([r16, r16 + N_NODES], axis=0)
    cols_seg = colp.reshape(16, SEG_CHUNKS, CHUNK)
    cols_deg = colp.reshape(32, DEG_CHUNKS, CHUNK)

    ones128 = jnp.ones((CHUNK, HALF), jnp.float32)
    zeros_big = jnp.zeros((NPAD, HALF), jnp.float32)

    deg2 = _deg_kernel(cols_deg, ones128, zeros_big)

    s1 = _mm1_call(deg2, x, W1)
    t1 = _seg_kernel(s1, rows_seg, cols_seg, zeros_big)
    s1_r = s1.reshape(2, N_NODES, HALF)

    s2 = _comb_call(deg2, t1, s1_r, b1.reshape(1, DIM), W2)
    t2 = _seg_kernel(s2, rows_seg, cols_seg, zeros_big)
    s2_r = s2.reshape(2, N_NODES, HALF)

    out = _fin_call(wt_idx.astype(jnp.int32), mut_idx.astype(jnp.int32),
                    deg2, t2, s2_r, b2.reshape(1, DIM),
                    mut_mask.reshape(10, _BLK), aa_emb, Wh1,
                    bh1.reshape(1, DIM), Wh2, bh2.reshape(1, 1))
    return out[0, 0]


# speedup vs baseline: 8.3792x; 1.1928x over previous
import functools

import jax
import jax.numpy as jnp
from jax import lax
from jax.experimental import pallas as pl
from jax.experimental.pallas import tpu as pltpu
from jax.experimental.pallas import tpu_sc as plsc

N_NODES = 10000
N_EDGES = 160000
DIM = 256
HALF = 128

NPAD = 10112
JUNK_ROW = 10016
EPAD = 163840
SEG_CHUNKS = 80
DEG_CHUNKS = 40
CHUNK = 128
ROWS_PER_TILE = NPAD // 16


def _deg_body(cols_hbm, ones_hbm, zeros_hbm, deg_hbm, colv, ones_v, acc, sem):
    c = lax.axis_index("c")
    s = lax.axis_index("s")
    tile = c * 16 + s
    pltpu.sync_copy(cols_hbm.at[tile], colv)
    pltpu.sync_copy(ones_hbm, ones_v)
    pltpu.sync_copy(
        zeros_hbm.at[pl.ds(s * ROWS_PER_TILE, ROWS_PER_TILE)],
        acc.at[pl.ds(s * ROWS_PER_TILE, ROWS_PER_TILE)],
    )
    plsc.subcore_barrier()

    @pl.loop(0, DEG_CHUNKS)
    def _(j):
        pltpu.sync_copy(ones_v, acc.at[colv.at[j]], add=True)

    plsc.subcore_barrier()
    pltpu.sync_copy(
        acc.at[pl.ds(s * ROWS_PER_TILE, ROWS_PER_TILE)],
        deg_hbm.at[c, pl.ds(s * ROWS_PER_TILE, ROWS_PER_TILE)],
    )


_deg_kernel = pl.kernel(
    _deg_body,
    out_type=jax.ShapeDtypeStruct((2, NPAD, HALF), jnp.float32),
    mesh=plsc.VectorSubcoreMesh(core_axis_name="c", subcore_axis_name="s"),
    scratch_types=[
        pltpu.VMEM((DEG_CHUNKS, CHUNK), jnp.int32),
        pltpu.VMEM((CHUNK, HALF), jnp.float32),
        pltpu.VMEM_SHARED((NPAD, HALF), jnp.float32),
        pltpu.SemaphoreType.DMA,
    ],
)


NBUF = 2
N_PASS = 2
PASS_CHUNKS = SEG_CHUNKS // N_PASS


def _seg_body(s_hbm, rows_hbm, cols_hbm, zeros_hbm, t_hbm,
              rowv, colv, buf0, buf1, acc, sem0, sem1):
    c = lax.axis_index("c")
    s = lax.axis_index("s")
    tile = c * 16 + s
    bufs = (buf0, buf1)
    sems = (sem0, sem1)
    pltpu.sync_copy(
        zeros_hbm.at[pl.ds(s * ROWS_PER_TILE, ROWS_PER_TILE)],
        acc.at[pl.ds(s * ROWS_PER_TILE, ROWS_PER_TILE)],
    )
    plsc.subcore_barrier()

    @pl.loop(0, N_PASS)
    def _(p):
        pltpu.sync_copy(
            rows_hbm.at[tile, pl.ds(p * PASS_CHUNKS, PASS_CHUNKS)], rowv)
        pltpu.sync_copy(
            cols_hbm.at[s, pl.ds(p * PASS_CHUNKS, PASS_CHUNKS)], colv)

        for b in range(NBUF):
            pltpu.async_copy(s_hbm.at[rowv.at[b]], bufs[b], sems[b])

        @pl.loop(0, PASS_CHUNKS - NBUF, step=NBUF)
        def _(j):
            for b in range(NBUF):
                pltpu.make_async_copy(
                    s_hbm.at[rowv.at[b]], bufs[b], sems[b]).wait()
                pltpu.sync_copy(bufs[b], acc.at[colv.at[j + b]], add=True)
                pltpu.async_copy(
                    s_hbm.at[rowv.at[j + NBUF + b]], bufs[b], sems[b])

        for b in range(NBUF):
            pltpu.make_async_copy(
                s_hbm.at[rowv.at[b]], bufs[b], sems[b]).wait()
            pltpu.sync_copy(bufs[b], acc.at[colv.at[PASS_CHUNKS - NBUF + b]],
                            add=True)

    plsc.subcore_barrier()
    pltpu.sync_copy(
        acc.at[pl.ds(s * ROWS_PER_TILE, ROWS_PER_TILE)],
        t_hbm.at[c, pl.ds(s * ROWS_PER_TILE, ROWS_PER_TILE)],
    )


_seg_kernel = pl.kernel(
    _seg_body,
    out_type=jax.ShapeDtypeStruct((2, NPAD, HALF), jnp.float32),
    mesh=plsc.VectorSubcoreMesh(core_axis_name="c", subcore_axis_name="s"),
    scratch_types=[
        pltpu.VMEM((PASS_CHUNKS, CHUNK), jnp.int32),
        pltpu.VMEM((PASS_CHUNKS, CHUNK), jnp.int32),
        pltpu.VMEM((CHUNK, HALF), jnp.float32),
        pltpu.VMEM((CHUNK, HALF), jnp.float32),
        pltpu.VMEM_SHARED((NPAD, HALF), jnp.float32),
        pltpu.SemaphoreType.DMA,
        pltpu.SemaphoreType.DMA,
    ],
)


def _dis_from(deg_ref):
    deg = deg_ref[0, :, 0:1] + deg_ref[1, :, 0:1] + 1.0
    return lax.rsqrt(deg)


def _mm1_body(x_ref, w_ref, o_ref):
    o_ref[...] = jnp.dot(x_ref[...], w_ref[...],
                         preferred_element_type=jnp.float32)


def _scale_body(deg_ref, u_ref, o_ref):
    o_ref[...] = _dis_from(deg_ref) * u_ref[...]


def _comb_body(deg_ref, t_ref, s_ref, b_ref, w_ref, o_ref):
    dis = _dis_from(deg_ref)
    ha = jnp.maximum(dis * (t_ref[0] + s_ref[0]) + b_ref[:, :HALF], 0.0)
    hb = jnp.maximum(dis * (t_ref[1] + s_ref[1]) + b_ref[:, HALF:], 0.0)
    h = jnp.concatenate([ha, hb], axis=1)
    o_ref[...] = dis * jnp.dot(h, w_ref[...], preferred_element_type=jnp.float32)


def _fin_body(wt_ref, mut_ref, deg_ref, t_ref, s_ref, b_ref, mask_ref,
              aa_ref, wh1_ref, bh1_ref, wh2_ref, bh2_ref, o_ref, zacc):
    i = pl.program_id(0)
    dis = _dis_from(deg_ref)
    ha = jnp.maximum(dis * (t_ref[0] + s_ref[0]) + b_ref[:, :HALF], 0.0)
    hb = jnp.maximum(dis * (t_ref[1] + s_ref[1]) + b_ref[:, HALF:], 0.0)
    h = jnp.concatenate([ha, hb], axis=1)
    part = jnp.dot(mask_ref[pl.ds(i, 1), :], h,
                   preferred_element_type=jnp.float32)

    @pl.when(i == 0)
    def _():
        zacc[...] = part

    @pl.when(i != 0)
    def _():
        zacc[...] = zacc[...] + part

    @pl.when(i == pl.num_programs(0) - 1)
    def _():
        z = zacc[...]
        wt = wt_ref[0]
        mut = mut_ref[0]
        delta = aa_ref[pl.ds(mut, 1), :] - aa_ref[pl.ds(wt, 1), :]
        hid = (jnp.dot(z, wh1_ref[:DIM, :], preferred_element_type=jnp.float32)
               + jnp.dot(delta, wh1_ref[DIM:, :],
                         preferred_element_type=jnp.float32)
               + bh1_ref[...])
        hid = jnp.maximum(hid, 0.0)
        o_ref[...] = jnp.dot(hid, wh2_ref[...],
                             preferred_element_type=jnp.float32) + bh2_ref[...]


_BLK = 1000


def _mm1_call(x, w1):
    return pl.pallas_call(
        _mm1_body,
        grid=(2, 10),
        in_specs=[
            pl.BlockSpec((_BLK, DIM), lambda j, i: (i, 0)),
            pl.BlockSpec((DIM, HALF), lambda j, i: (0, j)),
        ],
        out_specs=pl.BlockSpec((_BLK, HALF), lambda j, i: (j * 10 + i, 0)),
        out_shape=jax.ShapeDtypeStruct((2 * N_NODES, HALF), jnp.float32),
    )(x, w1)


def _scale_call(deg2, u):
    return pl.pallas_call(
        _scale_body,
        grid=(2, 10),
        in_specs=[
            pl.BlockSpec((2, _BLK, HALF), lambda j, i: (0, i, 0)),
            pl.BlockSpec((_BLK, HALF), lambda j, i: (j * 10 + i, 0)),
        ],
        out_specs=pl.BlockSpec((_BLK, HALF), lambda j, i: (j * 10 + i, 0)),
        out_shape=jax.ShapeDtypeStruct((2 * N_NODES, HALF), jnp.float32),
    )(deg2, u)


def _comb_call(deg2, t, s_r, b, w2):
    return pl.pallas_call(
        _comb_body,
        grid=(2, 10),
        in_specs=[
            pl.BlockSpec((2, _BLK, HALF), lambda j, i: (0, i, 0)),
            pl.BlockSpec((2, _BLK, HALF), lambda j, i: (0, i, 0)),
            pl.BlockSpec((2, _BLK, HALF), lambda j, i: (0, i, 0)),
            pl.BlockSpec((1, DIM), lambda j, i: (0, 0)),
            pl.BlockSpec((DIM, HALF), lambda j, i: (0, j)),
        ],
        out_specs=pl.BlockSpec((_BLK, HALF), lambda j, i: (j * 10 + i, 0)),
        out_shape=jax.ShapeDtypeStruct((2 * N_NODES, HALF), jnp.float32),
    )(deg2, t, s_r, b, w2)


def _fin_call(wt_idx, mut_idx, deg2, t, s_r, b, mask, aa, wh1, bh1, wh2, bh2):
    return pl.pallas_call(
        _fin_body,
        grid=(10,),
        in_specs=[
            pl.BlockSpec(memory_space=pltpu.MemorySpace.SMEM),
            pl.BlockSpec(memory_space=pltpu.MemorySpace.SMEM),
            pl.BlockSpec((2, _BLK, HALF), lambda i: (0, i, 0)),
            pl.BlockSpec((2, _BLK, HALF), lambda i: (0, i, 0)),
            pl.BlockSpec((2, _BLK, HALF), lambda i: (0, i, 0)),
            pl.BlockSpec((1, DIM), lambda i: (0, 0)),
            pl.BlockSpec((10, _BLK), lambda i: (0, 0)),
            pl.BlockSpec((20, 64), lambda i: (0, 0)),
            pl.BlockSpec((DIM + 64, DIM), lambda i: (0, 0)),
            pl.BlockSpec((1, DIM), lambda i: (0, 0)),
            pl.BlockSpec((DIM, 1), lambda i: (0, 0)),
            pl.BlockSpec((1, 1), lambda i: (0, 0)),
        ],
        out_specs=pl.BlockSpec((1, 1), lambda i: (0, 0)),
        out_shape=jax.ShapeDtypeStruct((1, 1), jnp.float32),
        scratch_shapes=[pltpu.VMEM((1, DIM), jnp.float32)],
    )(wt_idx, mut_idx, deg2, t, s_r, b, mask, aa, wh1, bh1, wh2, bh2)


def kernel(x, edge_index, mut_mask, wt_idx, mut_idx,
           W1, b1, W2, b2, aa_emb, Wh1, bh1, Wh2, bh2):
    row = edge_index[0].astype(jnp.int32)
    col = edge_index[1].astype(jnp.int32)
    pad = EPAD - N_EDGES
    rowp = jnp.concatenate([row, jnp.zeros((pad,), jnp.int32)])
    colp = jnp.concatenate([col, jnp.full((pad,), JUNK_ROW, jnp.int32)])

    r16 = rowp.reshape(16, SEG_CHUNKS, CHUNK)
    rows_seg = jnp.concatenate([r16, r16 + N_NODES], axis=0)
    cols_seg = colp.reshape(16, SEG_CHUNKS, CHUNK)
    cols_deg = colp.reshape(32, DEG_CHUNKS, CHUNK)

    ones128 = jnp.ones((CHUNK, HALF), jnp.float32)
    zeros_big = jnp.zeros((NPAD, HALF), jnp.float32)

    deg2 = _deg_kernel(cols_deg, ones128, zeros_big)

    u1 = _mm1_call(x, W1)
    s1 = _scale_call(deg2, u1)
    t1 = _seg_kernel(s1, rows_seg, cols_seg, zeros_big)
    s1_r = s1.reshape(2, N_NODES, HALF)

    s2 = _comb_call(deg2, t1, s1_r, b1.reshape(1, DIM), W2)
    t2 = _seg_kernel(s2, rows_seg, cols_seg, zeros_big)
    s2_r = s2.reshape(2, N_NODES, HALF)

    out = _fin_call(wt_idx.astype(jnp.int32), mut_idx.astype(jnp.int32),
                    deg2, t2, s2_r, b2.reshape(1, DIM),
                    mut_mask.reshape(10, _BLK), aa_emb, Wh1,
                    bh1.reshape(1, DIM), Wh2, bh2.reshape(1, 1))
    return out[0, 0]
